# baseline (speedup 1.0000x reference)
import jax
import jax.numpy as jnp
from jax.experimental import pallas as pl
from jax.experimental.pallas import tpu as pltpu

f32 = jnp.float32
bf16 = jnp.bfloat16
DN = (((1,), (0,)), ((), ()))


def _fold8(acc):
    acc = acc[:4] + acc[4:]
    acc = acc[:2] + acc[2:]
    return acc[0] + acc[1]


def _bn_relu(y, g, b):
    H, W, B, C = y.shape
    n = H * W * B
    s = jnp.sum(y.reshape(n, C), axis=0)
    mu = s * (1.0 / n)
    cent = y - mu[None, None, None, :]
    sq = cent * cent
    acc = jnp.zeros((B, C), f32)
    for w in range(W):
        for h in range(H):
            acc = acc + sq[h, w]
    var = _fold8(acc) * (1.0 / n)
    xn = cent / jnp.sqrt(var + 1e-5)[None, None, None, :]
    return jnp.maximum(xn * g[None, None, None, :] + b[None, None, None, :],
                       0.0)


def _pad_hw(a):
    H, W, B, C = a.shape
    z = jnp.zeros((1, W, B, C), f32)
    a = jnp.concatenate([z, a, z], axis=0)
    z = jnp.zeros((H + 2, 1, B, C), f32)
    return jnp.concatenate([z, a, z], axis=1)


def _im2col3x3(a):
    ap = _pad_hw(a)
    cols = [ap[ky:ky + 8, kx:kx + 8].reshape(512, 256)
            for ky in range(3) for kx in range(3)]
    return jnp.concatenate(cols, axis=1)


def _k_conv1(x36_ref, w_ref, b_ref, g_ref, be_ref, h1_ref):
    xr = x36_ref[...].reshape(18, 2, 18, 2, 8, 3)
    cols = [xr[ky // 2:ky // 2 + 16, ky % 2, kx // 2:kx // 2 + 16, kx % 2]
            .reshape(2048, 3) for ky in range(4) for kx in range(4)]
    pm = jnp.concatenate(cols, axis=1)
    c1 = jax.lax.dot_general(pm, w_ref[...], DN, preferred_element_type=f32)
    c1 = (c1 + b_ref[...][None, :]).reshape(16, 16, 8, 256)
    h1_ref[...] = _bn_relu(c1, g_ref[...], be_ref[...])


def _k_conv2(h1_ref, w_ref, b_ref, g_ref, be_ref, c2_ref, a1_ref):
    hp = _pad_hw(h1_ref[...]).reshape(9, 2, 9, 2, 8, 256)
    cols = [hp[ky // 2:ky // 2 + 8, ky % 2, kx // 2:kx // 2 + 8, kx % 2]
            .reshape(512, 256) for ky in range(4) for kx in range(4)]
    pm = jnp.concatenate(cols, axis=1)
    c2 = jax.lax.dot_general(pm, w_ref[...], DN, preferred_element_type=f32)
    c2 = (c2 + b_ref[...][None, :]).reshape(8, 8, 8, 256)
    c2_ref[...] = c2
    a1_ref[...] = _bn_relu(c2, g_ref[...], be_ref[...])


def _k_conv_mid(a_ref, w_ref, b_ref, g_ref, be_ref, o_ref):
    pm = _im2col3x3(a_ref[...])
    y = jax.lax.dot_general(pm, w_ref[...], DN, preferred_element_type=f32)
    y = (y + b_ref[...][None, :]).reshape(8, 8, 8, 256)
    o_ref[...] = _bn_relu(y, g_ref[...], be_ref[...])


def _k_conv_res(bq_ref, x_ref, w_ref, b_ref, g_ref, be_ref, h_ref, a_ref):
    pm = _im2col3x3(bq_ref[...])
    y = jax.lax.dot_general(pm, w_ref[...], DN, preferred_element_type=f32)
    h = (y + b_ref[...][None, :]).reshape(8, 8, 8, 256) + x_ref[...]
    h_ref[...] = h
    a_ref[...] = _bn_relu(h, g_ref[...], be_ref[...])


def _k_conv_ze(bq_ref, x_ref, w_ref, b_ref, ze_ref):
    pm = _im2col3x3(bq_ref[...])
    y = jax.lax.dot_general(pm, w_ref[...], DN, preferred_element_type=f32)
    ze_ref[...] = (y + b_ref[...][None, :]).reshape(8, 8, 8, 256) + x_ref[...]


def _k_vq(ze_ref, cb_ref, cbh_ref, cbm_ref, cbl_ref,
          qi_ref, zq_ref, vq_ref, dist_scr):
    cb = cb_ref[...]
    ze2d = ze_ref[...]

    def body(p, _):
        zrow = ze_ref[pl.ds(p, 1), :]
        diff = zrow - cb
        sq = diff * diff
        dtot = None
        for c in range(2):
            t = jnp.transpose(sq[:, 128 * c:128 * (c + 1)], (1, 0))
            acc = jnp.zeros((8, 512), f32)
            for u in range(16):
                acc = acc + t[8 * u:8 * u + 8]
            part = _fold8(acc)
            dtot = part if dtot is None else dtot + part
        dist_scr[pl.ds(p, 1), :] = dtot[None, :]
        return 0

    jax.lax.fori_loop(0, 512, body, 0)

    dist = dist_scr[...]
    amin = jnp.min(dist, axis=-1, keepdims=True)
    iota = jax.lax.broadcasted_iota(jnp.int32, (512, 512), 1)
    qi = jnp.min(jnp.where(dist == amin, iota, 512), axis=-1)
    qi_ref[...] = qi[:, None]

    oh = (iota == qi[:, None]).astype(f32)
    rows = jax.lax.dot_general(oh, cbh_ref[...], DN, preferred_element_type=f32)
    rows = rows + jax.lax.dot_general(oh, cbm_ref[...], DN,
                                      preferred_element_type=f32)
    rows = rows + jax.lax.dot_general(oh, cbl_ref[...], DN,
                                      preferred_element_type=f32)
    zq_ref[...] = ze2d + (rows - ze2d)
    dv = ze2d - rows
    vq_ref[...] = (jnp.sum(dv * dv) * (1.0 / (512.0 * 256.0)))[None, None]


def _conv3x3_dec(a, wflat, bias):
    pm = _im2col3x3(a)
    y = jax.lax.dot_general(pm, wflat, DN, preferred_element_type=f32)
    return (y + bias[None, :]).reshape(8, 8, 8, 256)


def _resblock_dec(x, wf1, b1, g1, be1, wf2, b2, g2, be2):
    h = _bn_relu(x, g1, be1)
    h = _conv3x3_dec(h, wf1, b1)
    h = _bn_relu(h, g2, be2)
    h = _conv3x3_dec(h, wf2, b2)
    return h + x


def _k_decoder(zq_ref, xh_ref,
               dr1w1_ref, dr1b1_ref, dr1g1_ref, dr1be1_ref,
               dr1w2_ref, dr1b2_ref, dr1g2_ref, dr1be2_ref,
               dr2w1_ref, dr2b1_ref, dr2g1_ref, dr2be1_ref,
               dr2w2_ref, dr2b2_ref, dr2g2_ref, dr2be2_ref,
               dbn1g_ref, dbn1b_ref, dbn2g_ref, dbn2b_ref,
               ct1w_ref, ct1b_ref, ct2w_ref, ct2b_ref,
               outs_ref, recon_ref):
    zq = zq_ref[...]
    d1 = _resblock_dec(zq, dr1w1_ref[...], dr1b1_ref[...], dr1g1_ref[...],
                       dr1be1_ref[...], dr1w2_ref[...], dr1b2_ref[...],
                       dr1g2_ref[...], dr1be2_ref[...])
    d2 = _resblock_dec(d1, dr2w1_ref[...], dr2b1_ref[...], dr2g1_ref[...],
                       dr2be1_ref[...], dr2w2_ref[...], dr2b2_ref[...],
                       dr2g2_ref[...], dr2be2_ref[...])
    db = _bn_relu(d2, dbn1g_ref[...], dbn1b_ref[...])

    dbp = _pad_hw(db)
    ct1w = ct1w_ref[...]
    ct1b = ct1b_ref[...]
    parts = []
    for a in range(2):
        for b in range(2):
            cols = [dbp[a + dy:a + dy + 8, b + dx:b + dx + 8].reshape(512, 256)
                    for dy in range(2) for dx in range(2)]
            pm = jnp.concatenate(cols, axis=1)
            r = jax.lax.dot_general(pm, ct1w[2 * a + b], DN,
                                    preferred_element_type=f32)
            parts.append((r + ct1b[None, :]).reshape(8, 1, 8, 1, 8, 256))
    up = jnp.concatenate(
        [jnp.concatenate([parts[0], parts[1]], axis=3),
         jnp.concatenate([parts[2], parts[3]], axis=3)], axis=1)
    d3 = _bn_relu(up.reshape(16, 16, 8, 256), dbn2g_ref[...], dbn2b_ref[...])

    d3p = _pad_hw(d3)
    ct2w = ct2w_ref[...]
    ct2b = ct2b_ref[...]
    parts = []
    for a in range(2):
        for b in range(2):
            cols = [d3p[a + dy:a + dy + 16, b + dx:b + dx + 16]
                    .reshape(2048, 256) for dy in range(2) for dx in range(2)]
            pm = jnp.concatenate(cols, axis=1)
            r = jax.lax.dot_general(pm, ct2w[2 * a + b], DN,
                                    preferred_element_type=f32)
            parts.append((r + ct2b[None, :]).reshape(16, 1, 16, 1, 8, 3))
    outs = jnp.concatenate(
        [jnp.concatenate([parts[0], parts[1]], axis=3),
         jnp.concatenate([parts[2], parts[3]], axis=3)], axis=1)
    outs = outs.reshape(32, 32, 8, 3)
    outs_ref[...] = outs
    rd = xh_ref[...] - outs
    recon_ref[...] = (jnp.sum(rd * rd)
                      * (1.0 / (8.0 * 3.0 * 32.0 * 32.0)))[None, None]


def _wflat(w):
    kh, kw = w.shape[2], w.shape[3]
    return (jnp.transpose(w, (2, 3, 1, 0))
            .reshape(kh * kw * w.shape[1], w.shape[0]).astype(bf16))


def _ctflat(w):
    mats = []
    for a in range(2):
        for b in range(2):
            taps = []
            for dy in range(2):
                for dx in range(2):
                    taps.append(jnp.transpose(w[:, :, 2 * dy + a, 2 * dx + b],
                                              (1, 0)))
            mats.append(jnp.concatenate(taps, axis=0))
    return jnp.stack(mats).astype(bf16)


def _call(fn, out_shapes, *args, scratch=None):
    return pl.pallas_call(fn, out_shape=out_shapes,
                          scratch_shapes=scratch or [])(*args)


def kernel(x, params):
    p = params
    xh = jnp.transpose(x, (1, 2, 0, 3)).astype(f32)
    x36 = jnp.pad(xh, ((1, 3), (1, 3), (0, 0), (0, 0)))

    cb = p['code_book']
    cbh = cb.astype(bf16)
    cbm = (cb - cbh.astype(f32)).astype(bf16)
    cbl = (cb - cbh.astype(f32) - cbm.astype(f32)).astype(bf16)

    sh_16 = jax.ShapeDtypeStruct((16, 16, 8, 256), f32)
    sh_8 = jax.ShapeDtypeStruct((8, 8, 8, 256), f32)

    h1 = _call(_k_conv1, sh_16, x36,
               _wflat(p['enc_c1_w']), p['enc_c1_b'],
               p['enc_bn1_g'], p['enc_bn1_b'])
    c2, a1 = _call(_k_conv2, (sh_8, sh_8), h1,
                   _wflat(p['enc_c2_w']), p['enc_c2_b'],
                   p['enc_r1']['bn1_g'], p['enc_r1']['bn1_b'])
    q = p['enc_r1']
    b1 = _call(_k_conv_mid, sh_8, a1, _wflat(q['c1_w']), q['c1_b'],
               q['bn2_g'], q['bn2_b'])
    q2 = p['enc_r2']
    h2, a2 = _call(_k_conv_res, (sh_8, sh_8), b1, c2,
                   _wflat(q['c2_w']), q['c2_b'], q2['bn1_g'], q2['bn1_b'])
    b2 = _call(_k_conv_mid, sh_8, a2, _wflat(q2['c1_w']), q2['c1_b'],
               q2['bn2_g'], q2['bn2_b'])
    ze = _call(_k_conv_ze, sh_8, b2, h2, _wflat(q2['c2_w']), q2['c2_b'])

    qi, zq2d, vq = _call(
        _k_vq,
        (jax.ShapeDtypeStruct((512, 1), jnp.int32),
         jax.ShapeDtypeStruct((512, 256), f32),
         jax.ShapeDtypeStruct((1, 1), f32)),
        ze.reshape(512, 256), cb, cbh, cbm, cbl,
        scratch=[pltpu.VMEM((512, 512), f32)])
    zq = zq2d.reshape(8, 8, 8, 256)

    outs, recon = _call(
        _k_decoder,
        (jax.ShapeDtypeStruct((32, 32, 8, 3), f32),
         jax.ShapeDtypeStruct((1, 1), f32)),
        zq, xh,
        _wflat(p['dec_r1']['c1_w']), p['dec_r1']['c1_b'],
        p['dec_r1']['bn1_g'], p['dec_r1']['bn1_b'],
        _wflat(p['dec_r1']['c2_w']), p['dec_r1']['c2_b'],
        p['dec_r1']['bn2_g'], p['dec_r1']['bn2_b'],
        _wflat(p['dec_r2']['c1_w']), p['dec_r2']['c1_b'],
        p['dec_r2']['bn1_g'], p['dec_r2']['bn1_b'],
        _wflat(p['dec_r2']['c2_w']), p['dec_r2']['c2_b'],
        p['dec_r2']['bn2_g'], p['dec_r2']['bn2_b'],
        p['dec_bn1_g'], p['dec_bn1_b'], p['dec_bn2_g'], p['dec_bn2_b'],
        _ctflat(p['dec_ct1_w']), p['dec_ct1_b'],
        _ctflat(p['dec_ct2_w']), p['dec_ct2_b'])

    q_index = jnp.transpose(qi.reshape(8, 8, 8), (2, 0, 1))
    outs_nchw = jnp.transpose(outs, (2, 3, 0, 1))
    recon_s = recon.reshape(())
    vq_s = vq.reshape(())
    return recon_s, vq_s, vq_s, q_index, outs_nchw


# speedup vs baseline: 1.2166x; 1.2166x over previous
import jax
import jax.numpy as jnp
from jax.experimental import pallas as pl
from jax.experimental.pallas import tpu as pltpu

f32 = jnp.float32
bf16 = jnp.bfloat16
DN = (((1,), (0,)), ((), ()))


def _fold8(acc):
    acc = acc[:4] + acc[4:]
    acc = acc[:2] + acc[2:]
    return acc[0] + acc[1]


def _bn_relu(y, g, b):
    H, W, B, C = y.shape
    n = H * W * B
    s = jnp.sum(y.reshape(n, C), axis=0)
    mu = s * (1.0 / n)
    cent = y - mu[None, None, None, :]
    sq = cent * cent
    acc = jnp.zeros((B, C), f32)
    for w in range(W):
        for h in range(H):
            acc = acc + sq[h, w]
    var = _fold8(acc) * (1.0 / n)
    xn = cent / jnp.sqrt(var + 1e-5)[None, None, None, :]
    return jnp.maximum(xn * g[None, None, None, :] + b[None, None, None, :],
                       0.0)


def _pad_hw(a):
    H, W, B, C = a.shape
    z = jnp.zeros((1, W, B, C), f32)
    a = jnp.concatenate([z, a, z], axis=0)
    z = jnp.zeros((H + 2, 1, B, C), f32)
    return jnp.concatenate([z, a, z], axis=1)


def _im2col3x3(a):
    ap = _pad_hw(a)
    cols = [ap[ky:ky + 8, kx:kx + 8].reshape(512, 256)
            for ky in range(3) for kx in range(3)]
    return jnp.concatenate(cols, axis=1)


def _k_conv1(x36_ref, w_ref, b_ref, g_ref, be_ref, h1_ref):
    xr = x36_ref[...].reshape(18, 2, 18, 2, 8, 3)
    cols = [xr[ky // 2:ky // 2 + 16, ky % 2, kx // 2:kx // 2 + 16, kx % 2]
            .reshape(2048, 3) for ky in range(4) for kx in range(4)]
    pm = jnp.concatenate(cols, axis=1)
    c1 = jax.lax.dot_general(pm, w_ref[...], DN, preferred_element_type=f32)
    c1 = (c1 + b_ref[...][None, :]).reshape(16, 16, 8, 256)
    h1_ref[...] = _bn_relu(c1, g_ref[...], be_ref[...])


def _k_conv2(h1_ref, w_ref, b_ref, g_ref, be_ref, c2_ref, a1_ref):
    hp = _pad_hw(h1_ref[...]).reshape(9, 2, 9, 2, 8, 256)
    cols = [hp[ky // 2:ky // 2 + 8, ky % 2, kx // 2:kx // 2 + 8, kx % 2]
            .reshape(512, 256) for ky in range(4) for kx in range(4)]
    pm = jnp.concatenate(cols, axis=1)
    c2 = jax.lax.dot_general(pm, w_ref[...], DN, preferred_element_type=f32)
    c2 = (c2 + b_ref[...][None, :]).reshape(8, 8, 8, 256)
    c2_ref[...] = c2
    a1_ref[...] = _bn_relu(c2, g_ref[...], be_ref[...])


def _k_conv_mid(a_ref, w_ref, b_ref, g_ref, be_ref, o_ref):
    pm = _im2col3x3(a_ref[...])
    y = jax.lax.dot_general(pm, w_ref[...], DN, preferred_element_type=f32)
    y = (y + b_ref[...][None, :]).reshape(8, 8, 8, 256)
    o_ref[...] = _bn_relu(y, g_ref[...], be_ref[...])


def _k_conv_res(bq_ref, x_ref, w_ref, b_ref, g_ref, be_ref, h_ref, a_ref):
    pm = _im2col3x3(bq_ref[...])
    y = jax.lax.dot_general(pm, w_ref[...], DN, preferred_element_type=f32)
    h = (y + b_ref[...][None, :]).reshape(8, 8, 8, 256) + x_ref[...]
    h_ref[...] = h
    a_ref[...] = _bn_relu(h, g_ref[...], be_ref[...])


def _k_conv_ze(bq_ref, x_ref, w_ref, b_ref, ze_ref):
    pm = _im2col3x3(bq_ref[...])
    y = jax.lax.dot_general(pm, w_ref[...], DN, preferred_element_type=f32)
    ze_ref[...] = (y + b_ref[...][None, :]).reshape(8, 8, 8, 256) + x_ref[...]


def _k_vq(ze_ref, cb_ref, cbh_ref, cbm_ref, cbl_ref,
          qi_ref, zq_ref, vq_ref, dist_scr):
    cb = cb_ref[...]
    ze2d = ze_ref[...]

    def body(i, _):
        zblk = ze_ref[pl.ds(8 * i, 8), :]
        diff = zblk[:, None, :] - cb[None, :, :]
        sq = diff * diff
        dtot = None
        for c in range(2):
            t = jnp.transpose(sq[:, :, 128 * c:128 * (c + 1)], (0, 2, 1))
            acc = jnp.zeros((8, 8, 512), f32)
            for u in range(16):
                acc = acc + t[:, 8 * u:8 * u + 8, :]
            acc = acc[:, :4] + acc[:, 4:]
            acc = acc[:, :2] + acc[:, 2:]
            part = acc[:, 0] + acc[:, 1]
            dtot = part if dtot is None else dtot + part
        dist_scr[pl.ds(8 * i, 8), :] = dtot
        return 0

    jax.lax.fori_loop(0, 64, body, 0)

    dist = dist_scr[...]
    amin = jnp.min(dist, axis=-1, keepdims=True)
    iota = jax.lax.broadcasted_iota(jnp.int32, (512, 512), 1)
    qi = jnp.min(jnp.where(dist == amin, iota, 512), axis=-1)
    qi_ref[...] = qi[:, None]

    oh = (iota == qi[:, None]).astype(f32)
    rows = jax.lax.dot_general(oh, cbh_ref[...], DN, preferred_element_type=f32)
    rows = rows + jax.lax.dot_general(oh, cbm_ref[...], DN,
                                      preferred_element_type=f32)
    rows = rows + jax.lax.dot_general(oh, cbl_ref[...], DN,
                                      preferred_element_type=f32)
    zq_ref[...] = ze2d + (rows - ze2d)
    dv = ze2d - rows
    vq_ref[...] = (jnp.sum(dv * dv) * (1.0 / (512.0 * 256.0)))[None, None]


def _conv3x3_dec(a, wflat, bias):
    pm = _im2col3x3(a)
    y = jax.lax.dot_general(pm, wflat, DN, preferred_element_type=f32)
    return (y + bias[None, :]).reshape(8, 8, 8, 256)


def _resblock_dec(x, wf1, b1, g1, be1, wf2, b2, g2, be2):
    h = _bn_relu(x, g1, be1)
    h = _conv3x3_dec(h, wf1, b1)
    h = _bn_relu(h, g2, be2)
    h = _conv3x3_dec(h, wf2, b2)
    return h + x


def _k_decoder(zq_ref, xh_ref,
               dr1w1_ref, dr1b1_ref, dr1g1_ref, dr1be1_ref,
               dr1w2_ref, dr1b2_ref, dr1g2_ref, dr1be2_ref,
               dr2w1_ref, dr2b1_ref, dr2g1_ref, dr2be1_ref,
               dr2w2_ref, dr2b2_ref, dr2g2_ref, dr2be2_ref,
               dbn1g_ref, dbn1b_ref, dbn2g_ref, dbn2b_ref,
               ct1w_ref, ct1b_ref, ct2w_ref, ct2b_ref,
               outs_ref, recon_ref):
    zq = zq_ref[...]
    d1 = _resblock_dec(zq, dr1w1_ref[...], dr1b1_ref[...], dr1g1_ref[...],
                       dr1be1_ref[...], dr1w2_ref[...], dr1b2_ref[...],
                       dr1g2_ref[...], dr1be2_ref[...])
    d2 = _resblock_dec(d1, dr2w1_ref[...], dr2b1_ref[...], dr2g1_ref[...],
                       dr2be1_ref[...], dr2w2_ref[...], dr2b2_ref[...],
                       dr2g2_ref[...], dr2be2_ref[...])
    db = _bn_relu(d2, dbn1g_ref[...], dbn1b_ref[...])

    dbp = _pad_hw(db)
    ct1w = ct1w_ref[...]
    ct1b = ct1b_ref[...]
    parts = []
    for a in range(2):
        for b in range(2):
            cols = [dbp[a + dy:a + dy + 8, b + dx:b + dx + 8].reshape(512, 256)
                    for dy in range(2) for dx in range(2)]
            pm = jnp.concatenate(cols, axis=1)
            r = jax.lax.dot_general(pm, ct1w[2 * a + b], DN,
                                    preferred_element_type=f32)
            parts.append((r + ct1b[None, :]).reshape(8, 1, 8, 1, 8, 256))
    up = jnp.concatenate(
        [jnp.concatenate([parts[0], parts[1]], axis=3),
         jnp.concatenate([parts[2], parts[3]], axis=3)], axis=1)
    d3 = _bn_relu(up.reshape(16, 16, 8, 256), dbn2g_ref[...], dbn2b_ref[...])

    d3p = _pad_hw(d3)
    ct2w = ct2w_ref[...]
    ct2b = ct2b_ref[...]
    parts = []
    for a in range(2):
        for b in range(2):
            cols = [d3p[a + dy:a + dy + 16, b + dx:b + dx + 16]
                    .reshape(2048, 256) for dy in range(2) for dx in range(2)]
            pm = jnp.concatenate(cols, axis=1)
            r = jax.lax.dot_general(pm, ct2w[2 * a + b], DN,
                                    preferred_element_type=f32)
            parts.append((r + ct2b[None, :]).reshape(16, 1, 16, 1, 8, 3))
    outs = jnp.concatenate(
        [jnp.concatenate([parts[0], parts[1]], axis=3),
         jnp.concatenate([parts[2], parts[3]], axis=3)], axis=1)
    outs = outs.reshape(32, 32, 8, 3)
    outs_ref[...] = outs
    rd = xh_ref[...] - outs
    recon_ref[...] = (jnp.sum(rd * rd)
                      * (1.0 / (8.0 * 3.0 * 32.0 * 32.0)))[None, None]


def _wflat(w):
    kh, kw = w.shape[2], w.shape[3]
    return (jnp.transpose(w, (2, 3, 1, 0))
            .reshape(kh * kw * w.shape[1], w.shape[0]).astype(bf16))


def _ctflat(w):
    mats = []
    for a in range(2):
        for b in range(2):
            taps = []
            for dy in range(2):
                for dx in range(2):
                    taps.append(jnp.transpose(w[:, :, 2 * dy + a, 2 * dx + b],
                                              (1, 0)))
            mats.append(jnp.concatenate(taps, axis=0))
    return jnp.stack(mats).astype(bf16)


def _call(fn, out_shapes, *args, scratch=None):
    return pl.pallas_call(fn, out_shape=out_shapes,
                          scratch_shapes=scratch or [])(*args)


def kernel(x, params):
    p = params
    xh = jnp.transpose(x, (1, 2, 0, 3)).astype(f32)
    x36 = jnp.pad(xh, ((1, 3), (1, 3), (0, 0), (0, 0)))

    cb = p['code_book']
    cbh = cb.astype(bf16)
    cbm = (cb - cbh.astype(f32)).astype(bf16)
    cbl = (cb - cbh.astype(f32) - cbm.astype(f32)).astype(bf16)

    sh_16 = jax.ShapeDtypeStruct((16, 16, 8, 256), f32)
    sh_8 = jax.ShapeDtypeStruct((8, 8, 8, 256), f32)

    h1 = _call(_k_conv1, sh_16, x36,
               _wflat(p['enc_c1_w']), p['enc_c1_b'],
               p['enc_bn1_g'], p['enc_bn1_b'])
    c2, a1 = _call(_k_conv2, (sh_8, sh_8), h1,
                   _wflat(p['enc_c2_w']), p['enc_c2_b'],
                   p['enc_r1']['bn1_g'], p['enc_r1']['bn1_b'])
    q = p['enc_r1']
    b1 = _call(_k_conv_mid, sh_8, a1, _wflat(q['c1_w']), q['c1_b'],
               q['bn2_g'], q['bn2_b'])
    q2 = p['enc_r2']
    h2, a2 = _call(_k_conv_res, (sh_8, sh_8), b1, c2,
                   _wflat(q['c2_w']), q['c2_b'], q2['bn1_g'], q2['bn1_b'])
    b2 = _call(_k_conv_mid, sh_8, a2, _wflat(q2['c1_w']), q2['c1_b'],
               q2['bn2_g'], q2['bn2_b'])
    ze = _call(_k_conv_ze, sh_8, b2, h2, _wflat(q2['c2_w']), q2['c2_b'])

    qi, zq2d, vq = _call(
        _k_vq,
        (jax.ShapeDtypeStruct((512, 1), jnp.int32),
         jax.ShapeDtypeStruct((512, 256), f32),
         jax.ShapeDtypeStruct((1, 1), f32)),
        ze.reshape(512, 256), cb, cbh, cbm, cbl,
        scratch=[pltpu.VMEM((512, 512), f32)])
    zq = zq2d.reshape(8, 8, 8, 256)

    outs, recon = _call(
        _k_decoder,
        (jax.ShapeDtypeStruct((32, 32, 8, 3), f32),
         jax.ShapeDtypeStruct((1, 1), f32)),
        zq, xh,
        _wflat(p['dec_r1']['c1_w']), p['dec_r1']['c1_b'],
        p['dec_r1']['bn1_g'], p['dec_r1']['bn1_b'],
        _wflat(p['dec_r1']['c2_w']), p['dec_r1']['c2_b'],
        p['dec_r1']['bn2_g'], p['dec_r1']['bn2_b'],
        _wflat(p['dec_r2']['c1_w']), p['dec_r2']['c1_b'],
        p['dec_r2']['bn1_g'], p['dec_r2']['bn1_b'],
        _wflat(p['dec_r2']['c2_w']), p['dec_r2']['c2_b'],
        p['dec_r2']['bn2_g'], p['dec_r2']['bn2_b'],
        p['dec_bn1_g'], p['dec_bn1_b'], p['dec_bn2_g'], p['dec_bn2_b'],
        _ctflat(p['dec_ct1_w']), p['dec_ct1_b'],
        _ctflat(p['dec_ct2_w']), p['dec_ct2_b'])

    q_index = jnp.transpose(qi.reshape(8, 8, 8), (2, 0, 1))
    outs_nchw = jnp.transpose(outs, (2, 3, 0, 1))
    recon_s = recon.reshape(())
    vq_s = vq.reshape(())
    return recon_s, vq_s, vq_s, q_index, outs_nchw


# speedup vs baseline: 1.2399x; 1.0191x over previous
import jax
import jax.numpy as jnp
from jax.experimental import pallas as pl
from jax.experimental.pallas import tpu as pltpu

f32 = jnp.float32
bf16 = jnp.bfloat16
DN = (((1,), (0,)), ((), ()))


def _fold8(acc):
    acc = acc[:4] + acc[4:]
    acc = acc[:2] + acc[2:]
    return acc[0] + acc[1]


def _bn_relu(y, g, b):
    H, W, B, C = y.shape
    n = H * W * B
    s = jnp.sum(y.reshape(n, C), axis=0)
    mu = s * (1.0 / n)
    cent = y - mu[None, None, None, :]
    sq = cent * cent
    acc = jnp.zeros((B, C), f32)
    for w in range(W):
        for h in range(H):
            acc = acc + sq[h, w]
    var = _fold8(acc) * (1.0 / n)
    xn = cent / jnp.sqrt(var + 1e-5)[None, None, None, :]
    return jnp.maximum(xn * g[None, None, None, :] + b[None, None, None, :],
                       0.0)


def _pad_hw(a):
    H, W, B, C = a.shape
    z = jnp.zeros((1, W, B, C), f32)
    a = jnp.concatenate([z, a, z], axis=0)
    z = jnp.zeros((H + 2, 1, B, C), f32)
    return jnp.concatenate([z, a, z], axis=1)


def _im2col3x3(a):
    ap = _pad_hw(a)
    cols = [ap[ky:ky + 8, kx:kx + 8].reshape(512, 256)
            for ky in range(3) for kx in range(3)]
    return jnp.concatenate(cols, axis=1)


def _k_conv1(x36_ref, w_ref, b_ref, g_ref, be_ref, h1_ref):
    xr = x36_ref[...].reshape(18, 2, 18, 2, 8, 3)
    cols = [xr[ky // 2:ky // 2 + 16, ky % 2, kx // 2:kx // 2 + 16, kx % 2]
            .reshape(2048, 3) for ky in range(4) for kx in range(4)]
    pm = jnp.concatenate(cols, axis=1)
    c1 = jax.lax.dot_general(pm, w_ref[...], DN, preferred_element_type=f32)
    c1 = (c1 + b_ref[...][None, :]).reshape(16, 16, 8, 256)
    h1_ref[...] = _bn_relu(c1, g_ref[...], be_ref[...])


def _k_conv2(h1_ref, w_ref, b_ref, g_ref, be_ref, c2_ref, a1_ref):
    hp = _pad_hw(h1_ref[...]).reshape(9, 2, 9, 2, 8, 256)
    cols = [hp[ky // 2:ky // 2 + 8, ky % 2, kx // 2:kx // 2 + 8, kx % 2]
            .reshape(512, 256) for ky in range(4) for kx in range(4)]
    pm = jnp.concatenate(cols, axis=1)
    c2 = jax.lax.dot_general(pm, w_ref[...], DN, preferred_element_type=f32)
    c2 = (c2 + b_ref[...][None, :]).reshape(8, 8, 8, 256)
    c2_ref[...] = c2
    a1_ref[...] = _bn_relu(c2, g_ref[...], be_ref[...])


def _k_conv_mid(a_ref, w_ref, b_ref, g_ref, be_ref, o_ref):
    pm = _im2col3x3(a_ref[...])
    y = jax.lax.dot_general(pm, w_ref[...], DN, preferred_element_type=f32)
    y = (y + b_ref[...][None, :]).reshape(8, 8, 8, 256)
    o_ref[...] = _bn_relu(y, g_ref[...], be_ref[...])


def _k_conv_res(bq_ref, x_ref, w_ref, b_ref, g_ref, be_ref, h_ref, a_ref):
    pm = _im2col3x3(bq_ref[...])
    y = jax.lax.dot_general(pm, w_ref[...], DN, preferred_element_type=f32)
    h = (y + b_ref[...][None, :]).reshape(8, 8, 8, 256) + x_ref[...]
    h_ref[...] = h
    a_ref[...] = _bn_relu(h, g_ref[...], be_ref[...])


def _k_conv_ze(bq_ref, x_ref, w_ref, b_ref, ze_ref):
    pm = _im2col3x3(bq_ref[...])
    y = jax.lax.dot_general(pm, w_ref[...], DN, preferred_element_type=f32)
    ze_ref[...] = (y + b_ref[...][None, :]).reshape(8, 8, 8, 256) + x_ref[...]


def _k_vq(ze_ref, cb_ref, cbh_ref, cbm_ref, cbl_ref,
          qi_ref, zq_ref, vq_ref, dist_scr):
    cb = cb_ref[...]
    ze2d = ze_ref[...]

    def body(i, _):
        zblk = ze_ref[pl.ds(16 * i, 16), :]
        diff = zblk[:, None, :] - cb[None, :, :]
        sq = diff * diff
        dtot = None
        for c in range(2):
            t = jnp.transpose(sq[:, :, 128 * c:128 * (c + 1)], (0, 2, 1))
            acc = jnp.zeros((16, 8, 512), f32)
            for u in range(16):
                acc = acc + t[:, 8 * u:8 * u + 8, :]
            acc = acc[:, :4] + acc[:, 4:]
            acc = acc[:, :2] + acc[:, 2:]
            part = acc[:, 0] + acc[:, 1]
            dtot = part if dtot is None else dtot + part
        dist_scr[pl.ds(16 * i, 16), :] = dtot
        return 0

    jax.lax.fori_loop(0, 32, body, 0)

    dist = dist_scr[...]
    amin = jnp.min(dist, axis=-1, keepdims=True)
    iota = jax.lax.broadcasted_iota(jnp.int32, (512, 512), 1)
    qi = jnp.min(jnp.where(dist == amin, iota, 512), axis=-1)
    qi_ref[...] = qi[:, None]

    oh = (iota == qi[:, None]).astype(f32)
    rows = jax.lax.dot_general(oh, cbh_ref[...], DN, preferred_element_type=f32)
    rows = rows + jax.lax.dot_general(oh, cbm_ref[...], DN,
                                      preferred_element_type=f32)
    rows = rows + jax.lax.dot_general(oh, cbl_ref[...], DN,
                                      preferred_element_type=f32)
    zq_ref[...] = ze2d + (rows - ze2d)
    dv = ze2d - rows
    vq_ref[...] = (jnp.sum(dv * dv) * (1.0 / (512.0 * 256.0)))[None, None]


def _conv3x3_dec(a, wflat, bias):
    pm = _im2col3x3(a)
    y = jax.lax.dot_general(pm, wflat, DN, preferred_element_type=f32)
    return (y + bias[None, :]).reshape(8, 8, 8, 256)


def _resblock_dec(x, wf1, b1, g1, be1, wf2, b2, g2, be2):
    h = _bn_relu(x, g1, be1)
    h = _conv3x3_dec(h, wf1, b1)
    h = _bn_relu(h, g2, be2)
    h = _conv3x3_dec(h, wf2, b2)
    return h + x


def _k_decoder(zq_ref, xh_ref,
               dr1w1_ref, dr1b1_ref, dr1g1_ref, dr1be1_ref,
               dr1w2_ref, dr1b2_ref, dr1g2_ref, dr1be2_ref,
               dr2w1_ref, dr2b1_ref, dr2g1_ref, dr2be1_ref,
               dr2w2_ref, dr2b2_ref, dr2g2_ref, dr2be2_ref,
               dbn1g_ref, dbn1b_ref, dbn2g_ref, dbn2b_ref,
               ct1w_ref, ct1b_ref, ct2w_ref, ct2b_ref,
               outs_ref, recon_ref):
    zq = zq_ref[...]
    d1 = _resblock_dec(zq, dr1w1_ref[...], dr1b1_ref[...], dr1g1_ref[...],
                       dr1be1_ref[...], dr1w2_ref[...], dr1b2_ref[...],
                       dr1g2_ref[...], dr1be2_ref[...])
    d2 = _resblock_dec(d1, dr2w1_ref[...], dr2b1_ref[...], dr2g1_ref[...],
                       dr2be1_ref[...], dr2w2_ref[...], dr2b2_ref[...],
                       dr2g2_ref[...], dr2be2_ref[...])
    db = _bn_relu(d2, dbn1g_ref[...], dbn1b_ref[...])

    dbp = _pad_hw(db)
    ct1w = ct1w_ref[...]
    ct1b = ct1b_ref[...]
    parts = []
    for a in range(2):
        for b in range(2):
            cols = [dbp[a + dy:a + dy + 8, b + dx:b + dx + 8].reshape(512, 256)
                    for dy in range(2) for dx in range(2)]
            pm = jnp.concatenate(cols, axis=1)
            r = jax.lax.dot_general(pm, ct1w[2 * a + b], DN,
                                    preferred_element_type=f32)
            parts.append((r + ct1b[None, :]).reshape(8, 1, 8, 1, 8, 256))
    up = jnp.concatenate(
        [jnp.concatenate([parts[0], parts[1]], axis=3),
         jnp.concatenate([parts[2], parts[3]], axis=3)], axis=1)
    d3 = _bn_relu(up.reshape(16, 16, 8, 256), dbn2g_ref[...], dbn2b_ref[...])

    d3p = _pad_hw(d3)
    ct2w = ct2w_ref[...]
    ct2b = ct2b_ref[...]
    parts = []
    for a in range(2):
        for b in range(2):
            cols = [d3p[a + dy:a + dy + 16, b + dx:b + dx + 16]
                    .reshape(2048, 256) for dy in range(2) for dx in range(2)]
            pm = jnp.concatenate(cols, axis=1)
            r = jax.lax.dot_general(pm, ct2w[2 * a + b], DN,
                                    preferred_element_type=f32)
            parts.append((r + ct2b[None, :]).reshape(16, 1, 16, 1, 8, 3))
    outs = jnp.concatenate(
        [jnp.concatenate([parts[0], parts[1]], axis=3),
         jnp.concatenate([parts[2], parts[3]], axis=3)], axis=1)
    outs = outs.reshape(32, 32, 8, 3)
    outs_ref[...] = outs
    rd = xh_ref[...] - outs
    recon_ref[...] = (jnp.sum(rd * rd)
                      * (1.0 / (8.0 * 3.0 * 32.0 * 32.0)))[None, None]


def _wflat(w):
    kh, kw = w.shape[2], w.shape[3]
    return (jnp.transpose(w, (2, 3, 1, 0))
            .reshape(kh * kw * w.shape[1], w.shape[0]).astype(bf16))


def _ctflat(w):
    mats = []
    for a in range(2):
        for b in range(2):
            taps = []
            for dy in range(2):
                for dx in range(2):
                    taps.append(jnp.transpose(w[:, :, 2 * dy + a, 2 * dx + b],
                                              (1, 0)))
            mats.append(jnp.concatenate(taps, axis=0))
    return jnp.stack(mats).astype(bf16)


def _call(fn, out_shapes, *args, scratch=None):
    return pl.pallas_call(fn, out_shape=out_shapes,
                          scratch_shapes=scratch or [])(*args)


def kernel(x, params):
    p = params
    xh = jnp.transpose(x, (1, 2, 0, 3)).astype(f32)
    x36 = jnp.pad(xh, ((1, 3), (1, 3), (0, 0), (0, 0)))

    cb = p['code_book']
    cbh = cb.astype(bf16)
    cbm = (cb - cbh.astype(f32)).astype(bf16)
    cbl = (cb - cbh.astype(f32) - cbm.astype(f32)).astype(bf16)

    sh_16 = jax.ShapeDtypeStruct((16, 16, 8, 256), f32)
    sh_8 = jax.ShapeDtypeStruct((8, 8, 8, 256), f32)

    h1 = _call(_k_conv1, sh_16, x36,
               _wflat(p['enc_c1_w']), p['enc_c1_b'],
               p['enc_bn1_g'], p['enc_bn1_b'])
    c2, a1 = _call(_k_conv2, (sh_8, sh_8), h1,
                   _wflat(p['enc_c2_w']), p['enc_c2_b'],
                   p['enc_r1']['bn1_g'], p['enc_r1']['bn1_b'])
    q = p['enc_r1']
    b1 = _call(_k_conv_mid, sh_8, a1, _wflat(q['c1_w']), q['c1_b'],
               q['bn2_g'], q['bn2_b'])
    q2 = p['enc_r2']
    h2, a2 = _call(_k_conv_res, (sh_8, sh_8), b1, c2,
                   _wflat(q['c2_w']), q['c2_b'], q2['bn1_g'], q2['bn1_b'])
    b2 = _call(_k_conv_mid, sh_8, a2, _wflat(q2['c1_w']), q2['c1_b'],
               q2['bn2_g'], q2['bn2_b'])
    ze = _call(_k_conv_ze, sh_8, b2, h2, _wflat(q2['c2_w']), q2['c2_b'])

    qi, zq2d, vq = _call(
        _k_vq,
        (jax.ShapeDtypeStruct((512, 1), jnp.int32),
         jax.ShapeDtypeStruct((512, 256), f32),
         jax.ShapeDtypeStruct((1, 1), f32)),
        ze.reshape(512, 256), cb, cbh, cbm, cbl,
        scratch=[pltpu.VMEM((512, 512), f32)])
    zq = zq2d.reshape(8, 8, 8, 256)

    outs, recon = _call(
        _k_decoder,
        (jax.ShapeDtypeStruct((32, 32, 8, 3), f32),
         jax.ShapeDtypeStruct((1, 1), f32)),
        zq, xh,
        _wflat(p['dec_r1']['c1_w']), p['dec_r1']['c1_b'],
        p['dec_r1']['bn1_g'], p['dec_r1']['bn1_b'],
        _wflat(p['dec_r1']['c2_w']), p['dec_r1']['c2_b'],
        p['dec_r1']['bn2_g'], p['dec_r1']['bn2_b'],
        _wflat(p['dec_r2']['c1_w']), p['dec_r2']['c1_b'],
        p['dec_r2']['bn1_g'], p['dec_r2']['bn1_b'],
        _wflat(p['dec_r2']['c2_w']), p['dec_r2']['c2_b'],
        p['dec_r2']['bn2_g'], p['dec_r2']['bn2_b'],
        p['dec_bn1_g'], p['dec_bn1_b'], p['dec_bn2_g'], p['dec_bn2_b'],
        _ctflat(p['dec_ct1_w']), p['dec_ct1_b'],
        _ctflat(p['dec_ct2_w']), p['dec_ct2_b'])

    q_index = jnp.transpose(qi.reshape(8, 8, 8), (2, 0, 1))
    outs_nchw = jnp.transpose(outs, (2, 3, 0, 1))
    recon_s = recon.reshape(())
    vq_s = vq.reshape(())
    return recon_s, vq_s, vq_s, q_index, outs_nchw
